# baseline (speedup 1.0000x reference)
import functools

import jax
import jax.numpy as jnp
from jax import lax
from jax.experimental import pallas as pl
from jax.experimental.pallas import tpu as pltpu
from jax.experimental.pallas import tpu_sc as plsc

N = 50000
NP = 50048
E = 800000
EP = 802816
EROWS = EP // 64
GP = 1152
CW = 32

_mesh = plsc.VectorSubcoreMesh(core_axis_name="c", subcore_axis_name="s")
_sc_params = pltpu.CompilerParams(use_tc_tiling_on_sc=False)


def _f32(*shape):
    return jax.ShapeDtypeStruct(shape, jnp.float32)


DW = 16


@functools.partial(
    pl.kernel,
    out_type=_f32(2 * NP, DW),
    mesh=_mesh,
    compiler_params=_sc_params,
    scratch_types=[
        pltpu.VMEM((28, 64), jnp.int32),
        pltpu.VMEM((64, DW), jnp.float32),
        pltpu.VMEM_SHARED((NP, DW), jnp.float32),
    ] + [pltpu.SemaphoreType.DMA] * 4,
)
def _deg_kernel(dst2d, ones64, zcol, deg_out, dstv, onesv, acc, *ss):
    c = lax.axis_index("c")
    s = lax.axis_index("s")
    pltpu.sync_copy(zcol.at[pl.ds(s * 3128, 3128)], acc.at[pl.ds(s * 3128, 3128)])
    pltpu.sync_copy(ones64, onesv)
    plsc.subcore_barrier()
    row0 = (c * 16 + s) * 392

    def bulk(b, _):
        pltpu.sync_copy(dst2d.at[pl.ds(row0 + b * 28, 28)], dstv)

        def quad(q, _):
            for t in range(4):
                pltpu.async_copy(onesv, acc.at[dstv.at[4 * q + t]], ss[t], add=True)
            for t in range(4):
                pltpu.make_async_copy(onesv, acc.at[dstv.at[4 * q + t]], ss[t]).wait()
            return 0

        lax.fori_loop(0, 7, quad, 0)
        return 0

    lax.fori_loop(0, 14, bulk, 0)
    plsc.subcore_barrier()
    pltpu.sync_copy(acc.at[pl.ds(s * 3128, 3128)],
                    deg_out.at[pl.ds(c * NP + s * 3128, 3128)])


def _zero_acc(zrows, acc, s):
    pltpu.sync_copy(zrows.at[pl.ds(s * 3128, 3128)], acc.at[pl.ds(s * 3128, 3128)])


def _edge_loop(gref, src2d, dst2d, sc, acc, row0, nbulks):
    (srcA, dstA, srcB, dstB, r0, r1, r2, r3,
     g0, g1, g2, g3, s0, s1, s2, s3, iA, iB) = sc
    rbufs = (r0, r1, r2, r3)
    gs = (g0, g1, g2, g3)
    ss = (s0, s1, s2, s3)

    def idx_copy(b, sv, dv, sem):
        pltpu.async_copy(src2d.at[pl.ds(row0 + b * 28, 28)], sv, sem)
        pltpu.async_copy(dst2d.at[pl.ds(row0 + b * 28, 28)], dv, sem)

    def idx_wait(b, sv, dv, sem):
        pltpu.make_async_copy(src2d.at[pl.ds(row0 + b * 28, 28)], sv, sem).wait()
        pltpu.make_async_copy(dst2d.at[pl.ds(row0 + b * 28, 28)], dv, sem).wait()

    def quad(sv, dv, J, fire_sv, fire_base):
        for t in range(4):
            pltpu.make_async_copy(gref.at[sv.at[J + t]], rbufs[t], gs[t]).wait()
            pltpu.async_copy(rbufs[t], acc.at[dv.at[J + t]], ss[t], add=True)
        for t in range(4):
            pltpu.make_async_copy(rbufs[t], acc.at[dv.at[J + t]], ss[t]).wait()
            if fire_sv is not None:
                pltpu.async_copy(gref.at[fire_sv.at[fire_base + t]], rbufs[t], gs[t])

    def main_quads(sv, dv):
        def body(k, _):
            quad(sv, dv, 4 * k, sv, 4 * k + 4)
            return 0
        lax.fori_loop(0, 6, body, 0)

    idx_copy(0, srcA, dstA, iA)
    idx_wait(0, srcA, dstA, iA)
    for t in range(4):
        pltpu.async_copy(gref.at[srcA.at[t]], rbufs[t], gs[t])

    nlast = nbulks // 2 - 1

    def bulk_pair(bp, _):
        idx_copy(2 * bp + 1, srcB, dstB, iB)
        main_quads(srcA, dstA)
        idx_wait(2 * bp + 1, srcB, dstB, iB)
        quad(srcA, dstA, 24, srcB, 0)

        @pl.when(bp < nlast)
        def _():
            idx_copy(2 * bp + 2, srcA, dstA, iA)

        main_quads(srcB, dstB)

        @pl.when(bp < nlast)
        def _():
            idx_wait(2 * bp + 2, srcA, dstA, iA)
            quad(srcB, dstB, 24, srcA, 0)

        @pl.when(bp == nlast)
        def _():
            quad(srcB, dstB, 24, None, 0)

        return 0

    lax.fori_loop(0, nbulks // 2, bulk_pair, 0)


_spmm_scratch = [
    pltpu.VMEM((28, 64), jnp.int32),
    pltpu.VMEM((28, 64), jnp.int32),
    pltpu.VMEM((28, 64), jnp.int32),
    pltpu.VMEM((28, 64), jnp.int32),
    pltpu.VMEM((64, CW), jnp.float32),
    pltpu.VMEM((64, CW), jnp.float32),
    pltpu.VMEM((64, CW), jnp.float32),
    pltpu.VMEM((64, CW), jnp.float32),
    pltpu.VMEM_SHARED((NP, CW), jnp.float32),
] + [pltpu.SemaphoreType.DMA] * 10


def _spmm_body(nchunks, args):
    gs = args[:nchunks]
    src2d, dst2d, zrows = args[nchunks:nchunks + 3]
    nouts = (nchunks + 1) // 2
    outs = args[nchunks + 3:nchunks + 3 + nouts]
    sc = args[nchunks + 3 + nouts:]
    acc = sc[8]
    scratch = sc[:8] + sc[9:]
    c = lax.axis_index("c")
    s = lax.axis_index("s")
    _zero_acc(zrows, acc, s)
    plsc.subcore_barrier()

    phases = []
    k = 0
    while k + 2 <= nchunks:
        phases.append((gs[k], gs[k + 1]))
        k += 2
    if k < nchunks:
        phases.append((gs[k],))

    for pi, ph in enumerate(phases):
        if len(ph) == 2:
            @pl.when(c == 0)
            def _(_ph=ph):
                _edge_loop(_ph[0], src2d, dst2d, scratch, acc, s * 784, 28)

            @pl.when(c == 1)
            def _(_ph=ph):
                _edge_loop(_ph[1], src2d, dst2d, scratch, acc, s * 784, 28)
        else:
            _edge_loop(ph[0], src2d, dst2d, scratch, acc,
                       (c * 16 + s) * 392, 14)
        plsc.subcore_barrier()
        pltpu.sync_copy(acc.at[pl.ds(s * 3128, 3128)],
                        outs[pi].at[pl.ds(c * NP + s * 3128, 3128)])
        if pi + 1 < len(phases):
            _zero_acc(zrows, acc, s)
            plsc.subcore_barrier()


def _make_spmm(nchunks):
    nouts = (nchunks + 1) // 2

    @functools.partial(
        pl.kernel,
        out_type=[_f32(2 * NP, CW) for _ in range(nouts)],
        mesh=_mesh,
        compiler_params=_sc_params,
        scratch_types=_spmm_scratch,
    )
    def k(*args):
        _spmm_body(nchunks, args)

    return k


_spmm3 = _make_spmm(3)
_spmm5 = _make_spmm(5)


@functools.partial(
    pl.kernel,
    out_type=_f32(2 * GP, 320),
    mesh=_mesh,
    compiler_params=_sc_params,
    scratch_types=[
        pltpu.VMEM((128, 320), jnp.float32),
        pltpu.VMEM((128,), jnp.int32),
        pltpu.VMEM_SHARED((GP, 320), jnp.float32),
        pltpu.SemaphoreType.DMA,
    ],
)
def _pool_kernel(h3, batch2d, zpool, pool_out, rowsv, bidxv, acc, sem):
    c = lax.axis_index("c")
    s = lax.axis_index("s")
    pltpu.sync_copy(zpool.at[pl.ds(s * 72, 72)], acc.at[pl.ds(s * 72, 72)])
    plsc.subcore_barrier()

    w = c * 16 + s
    nchunks = jnp.where(w < 7, 13, 12)

    def chunk(k, _):
        j = w + k * 32
        pltpu.sync_copy(batch2d.at[j], bidxv)
        pltpu.sync_copy(h3.at[pl.ds(j * 128, 128)], rowsv)
        pltpu.sync_copy(rowsv, acc.at[bidxv], add=True)
        return 0

    lax.fori_loop(0, nchunks, chunk, 0)
    plsc.subcore_barrier()
    pltpu.sync_copy(acc.at[pl.ds(s * 72, 72)],
                    pool_out.at[pl.ds(c * GP + s * 72, 72)])


BN = 3128
_GRID = (16,)


def _rows_spec(width, half=None):
    if half is None:
        return pl.BlockSpec((BN, width), lambda i: (i, 0))
    off = half * 16
    return pl.BlockSpec((BN, width), lambda i, _o=off: (i + _o, 0))


def _full_spec(shape):
    nd = len(shape)
    return pl.BlockSpec(shape, lambda *_: (0,) * nd)


def _nchunks(width):
    return (width + CW - 1) // CW


def _prep_body(x_ref, dega_ref, degb_ref, dinv_ref, *g_refs):
    deg = dega_ref[:, :1] + degb_ref[:, :1] + 1.0
    dinv = lax.rsqrt(deg)
    dinv_ref[...] = dinv
    g = x_ref[...] * dinv
    _write_chunks(g, 78, g_refs)


def _write_chunks(gn, width, outs):
    for k, oref in enumerate(outs):
        lo = k * CW
        hi = min(lo + CW, width)
        blk = gn[:, lo:hi]
        if hi - lo < CW:
            blk = jnp.concatenate(
                [blk, jnp.zeros((BN, CW - (hi - lo)), jnp.float32)], axis=1)
        oref[...] = blk


def _dense_body(pieces, width_in, width_out, last, *refs):
    nr = sum(len(p) for p in pieces)
    nci = len(pieces)
    rrefs = refs[:nr]
    gs = refs[nr:nr + nci]
    dinv_ref, w_ref, b_ref = refs[nr + nci:nr + nci + 3]
    outs = refs[nr + nci + 3:]
    cols = []
    for p in pieces:
        acc = rrefs[p[0]][...]
        for q in p[1:]:
            acc = acc + rrefs[q][...]
        cols.append(acc)
    r = jnp.concatenate(cols, axis=1)[:, :width_in]
    g = jnp.concatenate([ref[...] for ref in gs], axis=1)[:, :width_in]
    dinv = dinv_ref[...]
    ax = dinv * (r + g)
    h = jnp.maximum(jnp.dot(ax, w_ref[...],
                            preferred_element_type=jnp.float32) + b_ref[...], 0.0)
    if last:
        outs[0][...] = jnp.concatenate(
            [h, jnp.ones((BN, 1), jnp.float32),
             jnp.zeros((BN, 320 - width_out - 1), jnp.float32)], axis=1)
    else:
        _write_chunks(dinv * h, width_out, outs)


def _dense_layer(r_parts, g_chunks, dinv, W, b, width_in, width_out, last=False):
    nci = len(g_chunks)
    assert len(r_parts) == nci
    r_args, r_specs, pieces, idx = [], [], [], 0
    for arr, halves in r_parts:
        plist = []
        for h in halves:
            r_args.append(arr)
            r_specs.append(_rows_spec(CW, half=h))
            plist.append(idx)
            idx += 1
        pieces.append(plist)
    if last:
        out_shape = [_f32(NP, 320)]
        out_specs = [_rows_spec(320)]
    else:
        nco = _nchunks(width_out)
        out_shape = [_f32(NP, CW) for _ in range(nco)]
        out_specs = [_rows_spec(CW) for _ in range(nco)]
    in_specs = (
        r_specs
        + [_rows_spec(CW) for _ in range(nci)]
        + [_rows_spec(1), _full_spec(W.shape), _full_spec((1, width_out))]
    )
    body = functools.partial(_dense_body, pieces, width_in, width_out, last)
    outs = pl.pallas_call(
        body, grid=_GRID, in_specs=in_specs, out_specs=out_specs,
        out_shape=out_shape,
    )(*(r_args + list(g_chunks) + [dinv, W, b.reshape(1, -1)]))
    return outs


def _head_body(pool_ref, gemb_ref, wp_ref, bp_ref, wg_ref, bg_ref,
               wf1_ref, bf1_ref, wf2_ref, bf2_ref, o_ref):
    p = pool_ref[...]
    sums = p[:1024] + p[GP:GP + 1024]
    counts = sums[:, 312:313]
    x = sums[:, :312] / jnp.maximum(counts, 1.0)
    ge = jnp.dot(gemb_ref[...], wp_ref[...],
                 preferred_element_type=jnp.float32) + bp_ref[...]
    wg = wg_ref[...]
    gate = jax.nn.sigmoid(
        jnp.dot(x, wg[:312], preferred_element_type=jnp.float32)
        + jnp.dot(ge, wg[312:], preferred_element_type=jnp.float32)
        + bg_ref[...])
    fused = gate * ge + (1.0 - gate) * x
    h = jnp.maximum(jnp.dot(fused, wf1_ref[...],
                            preferred_element_type=jnp.float32) + bf1_ref[...], 0.0)
    o_ref[...] = jnp.dot(h, wf2_ref[...],
                         preferred_element_type=jnp.float32) + bf2_ref[...]


def _spmm_all(g_chunks, src2d, dst2d, zrows):
    nc = len(g_chunks)
    kern = _spmm3 if nc == 3 else _spmm5
    rs = kern(*g_chunks, src2d, dst2d, zrows)
    parts = []
    k = 0
    pi = 0
    while k + 2 <= nc:
        parts.append((rs[pi], [0]))
        parts.append((rs[pi], [1]))
        pi += 1
        k += 2
    if k < nc:
        parts.append((rs[pi], [0, 1]))
    return parts


def kernel(mol_x, mol_edge_index, mol_batch, global_emb, W1, b1, W2, b2,
           W3, b3, Wp, bp, Wg, bg, Wf1, bf1, Wf2, bf2):
    src = mol_edge_index[0].astype(jnp.int32)
    dst = mol_edge_index[1].astype(jnp.int32)
    batch = mol_batch.astype(jnp.int32)

    pad_e = EP - E
    src2d = jnp.concatenate([src, jnp.full((pad_e,), N, jnp.int32)]).reshape(EROWS, 64)
    dst2d = jnp.concatenate([dst, jnp.full((pad_e,), N, jnp.int32)]).reshape(EROWS, 64)
    batch2d = jnp.concatenate(
        [batch, jnp.full((NP - N,), 1024, jnp.int32)]).reshape(NP // 128, 128)
    x_pad = jnp.concatenate([mol_x, jnp.zeros((NP - N, 78), jnp.float32)])
    ones64 = jnp.ones((64, DW), jnp.float32)
    zcol = jnp.zeros((NP, DW), jnp.float32)
    zrows = jnp.zeros((NP, CW), jnp.float32)
    zpool = jnp.zeros((GP, 320), jnp.float32)

    degp = _deg_kernel(dst2d, ones64, zcol)
    prep = pl.pallas_call(
        _prep_body, grid=_GRID,
        in_specs=[_rows_spec(78), _rows_spec(DW, half=0), _rows_spec(DW, half=1)],
        out_specs=[_rows_spec(1)] + [_rows_spec(CW)] * 3,
        out_shape=[_f32(NP, 1)] + [_f32(NP, CW)] * 3,
    )(x_pad, degp, degp)
    dinv, g1 = prep[0], prep[1:]

    r1 = _spmm_all(g1, src2d, dst2d, zrows)
    g2 = _dense_layer(r1, g1, dinv, W1, b1, 78, 78)

    r2 = _spmm_all(g2, src2d, dst2d, zrows)
    g3 = _dense_layer(r2, g2, dinv, W2, b2, 78, 156)

    r3 = _spmm_all(g3, src2d, dst2d, zrows)
    (h3,) = _dense_layer(r3, g3, dinv, W3, b3, 156, 312, last=True)

    pool = _pool_kernel(h3, batch2d, zpool)
    out = pl.pallas_call(
        _head_body,
        in_specs=[_full_spec((2 * GP, 320)), _full_spec((1024, 128)),
                  _full_spec((128, 312)), _full_spec((1, 312)),
                  _full_spec((624, 1)), _full_spec((1, 1)),
                  _full_spec((312, 1024)), _full_spec((1, 1024)),
                  _full_spec((1024, 128)), _full_spec((1, 128))],
        out_specs=_full_spec((1024, 128)),
        out_shape=_f32(1024, 128),
    )(pool, global_emb, Wp, bp.reshape(1, -1), Wg, bg.reshape(1, -1),
      Wf1, bf1.reshape(1, -1), Wf2, bf2.reshape(1, -1))
    return out


# speedup vs baseline: 1.1288x; 1.1288x over previous
import functools

import jax
import jax.numpy as jnp
from jax import lax
from jax.experimental import pallas as pl
from jax.experimental.pallas import tpu as pltpu
from jax.experimental.pallas import tpu_sc as plsc

N = 50000
NP = 50048
E = 800000
EP = 802816
EROWS = EP // 64
GP = 1152
CW = 32

_mesh = plsc.VectorSubcoreMesh(core_axis_name="c", subcore_axis_name="s")
_sc_params = pltpu.CompilerParams(use_tc_tiling_on_sc=False)


def _f32(*shape):
    return jax.ShapeDtypeStruct(shape, jnp.float32)


DW = 16


@functools.partial(
    pl.kernel,
    out_type=_f32(2 * NP, DW),
    mesh=_mesh,
    compiler_params=_sc_params,
    scratch_types=[
        pltpu.VMEM((28, 64), jnp.int32),
        pltpu.VMEM((64, DW), jnp.float32),
        pltpu.VMEM_SHARED((NP, DW), jnp.float32),
    ] + [pltpu.SemaphoreType.DMA] * 4,
)
def _deg_kernel(dst2d, ones64, zcol, deg_out, dstv, onesv, acc, *ss):
    c = lax.axis_index("c")
    s = lax.axis_index("s")
    pltpu.sync_copy(zcol.at[pl.ds(s * 3128, 3128)], acc.at[pl.ds(s * 3128, 3128)])
    pltpu.sync_copy(ones64, onesv)
    plsc.subcore_barrier()
    row0 = (c * 16 + s) * 392

    def bulk(b, _):
        pltpu.sync_copy(dst2d.at[pl.ds(row0 + b * 28, 28)], dstv)

        def quad(q, _):
            for t in range(4):
                pltpu.async_copy(onesv, acc.at[dstv.at[4 * q + t]], ss[t], add=True)
            for t in range(4):
                pltpu.make_async_copy(onesv, acc.at[dstv.at[4 * q + t]], ss[t]).wait()
            return 0

        lax.fori_loop(0, 7, quad, 0)
        return 0

    lax.fori_loop(0, 14, bulk, 0)
    plsc.subcore_barrier()
    pltpu.sync_copy(acc.at[pl.ds(s * 3128, 3128)],
                    deg_out.at[pl.ds(c * NP + s * 3128, 3128)])


def _zero_acc(zrows, acc, s):
    pltpu.sync_copy(zrows.at[pl.ds(s * 3128, 3128)], acc.at[pl.ds(s * 3128, 3128)])


def _edge_loop(gref, src2d, dst2d, sc, acc, row0, nbulks):
    (srcA, dstA, srcB, dstB, r0, r1, r2, r3,
     g0, g1, g2, g3, s0, s1, s2, s3, iA, iB) = sc
    rbufs = (r0, r1, r2, r3)
    gs = (g0, g1, g2, g3)
    ss = (s0, s1, s2, s3)

    def idx_copy(b, sv, dv, sem):
        pltpu.async_copy(src2d.at[pl.ds(row0 + b * 28, 28)], sv, sem)
        pltpu.async_copy(dst2d.at[pl.ds(row0 + b * 28, 28)], dv, sem)

    def idx_wait(b, sv, dv, sem):
        pltpu.make_async_copy(src2d.at[pl.ds(row0 + b * 28, 28)], sv, sem).wait()
        pltpu.make_async_copy(dst2d.at[pl.ds(row0 + b * 28, 28)], dv, sem).wait()

    def quad(sv, dv, J, fire_sv, fire_base):
        for t in range(4):
            pltpu.make_async_copy(gref.at[sv.at[J + t]], rbufs[t], gs[t]).wait()
            pltpu.async_copy(rbufs[t], acc.at[dv.at[J + t]], ss[t], add=True)
        for t in range(4):
            pltpu.make_async_copy(rbufs[t], acc.at[dv.at[J + t]], ss[t]).wait()
            if fire_sv is not None:
                pltpu.async_copy(gref.at[fire_sv.at[fire_base + t]], rbufs[t], gs[t])

    def main_quads(sv, dv):
        def body(k, _):
            quad(sv, dv, 4 * k, sv, 4 * k + 4)
            return 0
        lax.fori_loop(0, 6, body, 0)

    idx_copy(0, srcA, dstA, iA)
    idx_wait(0, srcA, dstA, iA)
    for t in range(4):
        pltpu.async_copy(gref.at[srcA.at[t]], rbufs[t], gs[t])

    nlast = nbulks // 2 - 1

    def bulk_pair(bp, _):
        idx_copy(2 * bp + 1, srcB, dstB, iB)
        main_quads(srcA, dstA)
        idx_wait(2 * bp + 1, srcB, dstB, iB)
        quad(srcA, dstA, 24, srcB, 0)

        @pl.when(bp < nlast)
        def _():
            idx_copy(2 * bp + 2, srcA, dstA, iA)

        main_quads(srcB, dstB)

        @pl.when(bp < nlast)
        def _():
            idx_wait(2 * bp + 2, srcA, dstA, iA)
            quad(srcB, dstB, 24, srcA, 0)

        @pl.when(bp == nlast)
        def _():
            quad(srcB, dstB, 24, None, 0)

        return 0

    lax.fori_loop(0, nbulks // 2, bulk_pair, 0)


_spmm_scratch = [
    pltpu.VMEM((28, 64), jnp.int32),
    pltpu.VMEM((28, 64), jnp.int32),
    pltpu.VMEM((28, 64), jnp.int32),
    pltpu.VMEM((28, 64), jnp.int32),
    pltpu.VMEM((64, CW), jnp.float32),
    pltpu.VMEM((64, CW), jnp.float32),
    pltpu.VMEM((64, CW), jnp.float32),
    pltpu.VMEM((64, CW), jnp.float32),
    pltpu.VMEM_SHARED((NP, CW), jnp.float32),
] + [pltpu.SemaphoreType.DMA] * 10


def _spmm_body(nchunks, args):
    gs = args[:nchunks]
    src2d, dst2d, zrows = args[nchunks:nchunks + 3]
    nouts = (nchunks + 1) // 2
    outs = args[nchunks + 3:nchunks + 3 + nouts]
    sc = args[nchunks + 3 + nouts:]
    acc = sc[8]
    scratch = sc[:8] + sc[9:]
    c = lax.axis_index("c")
    s = lax.axis_index("s")
    _zero_acc(zrows, acc, s)
    plsc.subcore_barrier()

    phases = []
    k = 0
    while k + 2 <= nchunks:
        phases.append((gs[k], gs[k + 1]))
        k += 2
    if k < nchunks:
        phases.append((gs[k],))

    for pi, ph in enumerate(phases):
        if len(ph) == 2:
            @pl.when(c == 0)
            def _(_ph=ph):
                _edge_loop(_ph[0], src2d, dst2d, scratch, acc, s * 784, 28)

            @pl.when(c == 1)
            def _(_ph=ph):
                _edge_loop(_ph[1], src2d, dst2d, scratch, acc, s * 784, 28)
        else:
            _edge_loop(ph[0], src2d, dst2d, scratch, acc,
                       (c * 16 + s) * 392, 14)
        plsc.subcore_barrier()
        pltpu.sync_copy(acc.at[pl.ds(s * 3128, 3128)],
                        outs[pi].at[pl.ds(c * NP + s * 3128, 3128)])
        if pi + 1 < len(phases):
            _zero_acc(zrows, acc, s)
            plsc.subcore_barrier()


def _make_spmm(nchunks):
    nouts = (nchunks + 1) // 2

    @functools.partial(
        pl.kernel,
        out_type=[_f32(2 * NP, CW) for _ in range(nouts)],
        mesh=_mesh,
        compiler_params=_sc_params,
        scratch_types=_spmm_scratch,
    )
    def k(*args):
        _spmm_body(nchunks, args)

    return k


_spmm2 = _make_spmm(2)
_spmm1 = _make_spmm(1)


BN = 3128
_GRID = (16,)


def _rows_spec(width, half=None):
    if half is None:
        return pl.BlockSpec((BN, width), lambda i: (i, 0))
    off = half * 16
    return pl.BlockSpec((BN, width), lambda i, _o=off: (i + _o, 0))


def _full_spec(shape):
    nd = len(shape)
    return pl.BlockSpec(shape, lambda *_: (0,) * nd)


def _nchunks(width):
    return (width + CW - 1) // CW


def _prep_body(x_ref, dega_ref, degb_ref, dinv_ref, *g_refs):
    deg = dega_ref[:, :1] + degb_ref[:, :1] + 1.0
    dinv = lax.rsqrt(deg)
    dinv_ref[...] = dinv
    g = x_ref[...] * dinv
    _write_chunks(g, 78, g_refs)


def _write_chunks(gn, width, outs):
    for k, oref in enumerate(outs):
        lo = k * CW
        hi = min(lo + CW, width)
        blk = gn[:, lo:hi]
        if hi - lo < CW:
            blk = jnp.concatenate(
                [blk, jnp.zeros((BN, CW - (hi - lo)), jnp.float32)], axis=1)
        oref[...] = blk


def _dense_body(pieces, width_in, width_out, last, *refs):
    nr = sum(len(p) for p in pieces)
    nci = len(pieces)
    rrefs = refs[:nr]
    gs = refs[nr:nr + nci]
    dinv_ref, w_ref, b_ref = refs[nr + nci:nr + nci + 3]
    outs = refs[nr + nci + 3 + (1 if last else 0):]
    cols = []
    for p in pieces:
        acc = rrefs[p[0]][...]
        for q in p[1:]:
            acc = acc + rrefs[q][...]
        cols.append(acc)
    r = jnp.concatenate(cols, axis=1)[:, :width_in]
    g = jnp.concatenate([ref[...] for ref in gs], axis=1)[:, :width_in]
    dinv = dinv_ref[...]
    ax = dinv * (r + g)
    h = jnp.maximum(jnp.dot(ax, w_ref[...],
                            preferred_element_type=jnp.float32) + b_ref[...], 0.0)
    if last:
        batch_ref = refs[nr + nci + 3]
        h3 = jnp.concatenate(
            [h, jnp.ones((BN, 1), jnp.float32),
             jnp.zeros((BN, 320 - width_out - 1), jnp.float32)], axis=1)
        onehot = (batch_ref[...] ==
                  lax.broadcasted_iota(jnp.int32, (1, 1024), 1)
                  ).astype(jnp.float32)
        contrib = lax.dot_general(
            onehot, h3, (((0,), (0,)), ((), ())),
            preferred_element_type=jnp.float32)

        @pl.when(pl.program_id(0) == 0)
        def _():
            outs[0][...] = jnp.zeros((1024, 320), jnp.float32)

        outs[0][...] += contrib
    else:
        _write_chunks(dinv * h, width_out, outs)


def _dense_layer(r_parts, g_chunks, dinv, W, b, width_in, width_out,
                 last=False, batch_pad=None):
    nci = len(g_chunks)
    assert len(r_parts) == nci
    r_args, r_specs, pieces, idx = [], [], [], 0
    for arr, halves in r_parts:
        plist = []
        for h in halves:
            r_args.append(arr)
            r_specs.append(_rows_spec(CW, half=h))
            plist.append(idx)
            idx += 1
        pieces.append(plist)
    if last:
        out_shape = [_f32(1024, 320)]
        out_specs = [_full_spec((1024, 320))]
    else:
        nco = _nchunks(width_out)
        out_shape = [_f32(NP, CW) for _ in range(nco)]
        out_specs = [_rows_spec(CW) for _ in range(nco)]
    in_specs = (
        r_specs
        + [_rows_spec(CW) for _ in range(nci)]
        + [_rows_spec(1), _full_spec(W.shape), _full_spec((1, width_out))]
        + ([_rows_spec(1)] if last else [])
    )
    body = functools.partial(_dense_body, pieces, width_in, width_out, last)
    args = r_args + list(g_chunks) + [dinv, W, b.reshape(1, -1)]
    if last:
        args.append(batch_pad)
    outs = pl.pallas_call(
        body, grid=_GRID, in_specs=in_specs, out_specs=out_specs,
        out_shape=out_shape,
    )(*args)
    return outs


def _head_body(pool_ref, gemb_ref, wp_ref, bp_ref, wg_ref, bg_ref,
               wf1_ref, bf1_ref, wf2_ref, bf2_ref, o_ref):
    sums = pool_ref[...]
    counts = sums[:, 312:313]
    x = sums[:, :312] / jnp.maximum(counts, 1.0)
    ge = jnp.dot(gemb_ref[...], wp_ref[...],
                 preferred_element_type=jnp.float32) + bp_ref[...]
    wg = wg_ref[...]
    gate = jax.nn.sigmoid(
        jnp.dot(x, wg[:312], preferred_element_type=jnp.float32)
        + jnp.dot(ge, wg[312:], preferred_element_type=jnp.float32)
        + bg_ref[...])
    fused = gate * ge + (1.0 - gate) * x
    h = jnp.maximum(jnp.dot(fused, wf1_ref[...],
                            preferred_element_type=jnp.float32) + bf1_ref[...], 0.0)
    o_ref[...] = jnp.dot(h, wf2_ref[...],
                         preferred_element_type=jnp.float32) + bf2_ref[...]


def _spmm_all(g_chunks, src2d, dst2d, zrows):
    nc = len(g_chunks)
    parts = []
    k = 0
    while k + 2 <= nc:
        (r,) = _spmm2(g_chunks[k], g_chunks[k + 1], src2d, dst2d, zrows)
        parts.append((r, [0]))
        parts.append((r, [1]))
        k += 2
    if k < nc:
        (r,) = _spmm1(g_chunks[k], src2d, dst2d, zrows)
        parts.append((r, [0, 1]))
    return parts


def kernel(mol_x, mol_edge_index, mol_batch, global_emb, W1, b1, W2, b2,
           W3, b3, Wp, bp, Wg, bg, Wf1, bf1, Wf2, bf2):
    src = mol_edge_index[0].astype(jnp.int32)
    dst = mol_edge_index[1].astype(jnp.int32)
    batch = mol_batch.astype(jnp.int32)

    pad_e = EP - E
    src2d = jnp.concatenate([src, jnp.full((pad_e,), N, jnp.int32)]).reshape(EROWS, 64)
    dst2d = jnp.concatenate([dst, jnp.full((pad_e,), N, jnp.int32)]).reshape(EROWS, 64)
    batch_pad = jnp.concatenate(
        [batch, jnp.full((NP - N,), 1024, jnp.int32)]).reshape(NP, 1)
    x_pad = jnp.concatenate([mol_x, jnp.zeros((NP - N, 78), jnp.float32)])
    ones64 = jnp.ones((64, DW), jnp.float32)
    zcol = jnp.zeros((NP, DW), jnp.float32)
    zrows = jnp.zeros((NP, CW), jnp.float32)

    degp = _deg_kernel(dst2d, ones64, zcol)
    prep = pl.pallas_call(
        _prep_body, grid=_GRID,
        in_specs=[_rows_spec(78), _rows_spec(DW, half=0), _rows_spec(DW, half=1)],
        out_specs=[_rows_spec(1)] + [_rows_spec(CW)] * 3,
        out_shape=[_f32(NP, 1)] + [_f32(NP, CW)] * 3,
    )(x_pad, degp, degp)
    dinv, g1 = prep[0], prep[1:]

    r1 = _spmm_all(g1, src2d, dst2d, zrows)
    g2 = _dense_layer(r1, g1, dinv, W1, b1, 78, 78)

    r2 = _spmm_all(g2, src2d, dst2d, zrows)
    g3 = _dense_layer(r2, g2, dinv, W2, b2, 78, 156)

    r3 = _spmm_all(g3, src2d, dst2d, zrows)
    (pool,) = _dense_layer(r3, g3, dinv, W3, b3, 156, 312, last=True,
                           batch_pad=batch_pad)

    out = pl.pallas_call(
        _head_body,
        in_specs=[_full_spec((1024, 320)), _full_spec((1024, 128)),
                  _full_spec((128, 312)), _full_spec((1, 312)),
                  _full_spec((624, 1)), _full_spec((1, 1)),
                  _full_spec((312, 1024)), _full_spec((1, 1024)),
                  _full_spec((1024, 128)), _full_spec((1, 128))],
        out_specs=_full_spec((1024, 128)),
        out_shape=_f32(1024, 128),
    )(pool, global_emb, Wp, bp.reshape(1, -1), Wg, bg.reshape(1, -1),
      Wf1, bf1.reshape(1, -1), Wf2, bf2.reshape(1, -1))
    return out


# speedup vs baseline: 1.1691x; 1.0357x over previous
import functools

import jax
import jax.numpy as jnp
from jax import lax
from jax.experimental import pallas as pl
from jax.experimental.pallas import tpu as pltpu
from jax.experimental.pallas import tpu_sc as plsc

N = 50000
NP = 50048
E = 800000
EP = 802816
EROWS = EP // 64
GP = 1152
CW = 32

_mesh = plsc.VectorSubcoreMesh(core_axis_name="c", subcore_axis_name="s")
_sc_params = pltpu.CompilerParams(use_tc_tiling_on_sc=False)


def _f32(*shape):
    return jax.ShapeDtypeStruct(shape, jnp.float32)


DW = 16


@functools.partial(
    pl.kernel,
    out_type=_f32(2 * NP, DW),
    mesh=_mesh,
    compiler_params=_sc_params,
    scratch_types=[
        pltpu.VMEM((28, 64), jnp.int32),
        pltpu.VMEM((64, DW), jnp.float32),
        pltpu.VMEM_SHARED((NP, DW), jnp.float32),
    ] + [pltpu.SemaphoreType.DMA] * 4,
)
def _deg_kernel(dst2d, ones64, zcol, deg_out, dstv, onesv, acc, *ss):
    c = lax.axis_index("c")
    s = lax.axis_index("s")
    pltpu.sync_copy(zcol.at[pl.ds(s * 3128, 3128)], acc.at[pl.ds(s * 3128, 3128)])
    pltpu.sync_copy(ones64, onesv)
    plsc.subcore_barrier()
    row0 = (c * 16 + s) * 392

    def bulk(b, _):
        pltpu.sync_copy(dst2d.at[pl.ds(row0 + b * 28, 28)], dstv)

        def quad(q, _):
            for t in range(4):
                pltpu.async_copy(onesv, acc.at[dstv.at[4 * q + t]], ss[t], add=True)
            for t in range(4):
                pltpu.make_async_copy(onesv, acc.at[dstv.at[4 * q + t]], ss[t]).wait()
            return 0

        lax.fori_loop(0, 7, quad, 0)
        return 0

    lax.fori_loop(0, 14, bulk, 0)
    plsc.subcore_barrier()
    pltpu.sync_copy(acc.at[pl.ds(s * 3128, 3128)],
                    deg_out.at[pl.ds(c * NP + s * 3128, 3128)])


def _zero_acc(zrows, acc, s):
    pltpu.sync_copy(zrows.at[pl.ds(s * 3128, 3128)], acc.at[pl.ds(s * 3128, 3128)])


def _edge_loop(gref, src2d, dst2d, sc, acc, row0, nbulks):
    (srcA, dstA, srcB, dstB, r0, r1, r2, r3,
     g0, g1, g2, g3, s0, s1, s2, s3, iA, iB) = sc
    rbufs = (r0, r1, r2, r3)
    gs = (g0, g1, g2, g3)
    ss = (s0, s1, s2, s3)

    def idx_copy(b, sv, dv, sem):
        pltpu.async_copy(src2d.at[pl.ds(row0 + b * 28, 28)], sv, sem)
        pltpu.async_copy(dst2d.at[pl.ds(row0 + b * 28, 28)], dv, sem)

    def idx_wait(b, sv, dv, sem):
        pltpu.make_async_copy(src2d.at[pl.ds(row0 + b * 28, 28)], sv, sem).wait()
        pltpu.make_async_copy(dst2d.at[pl.ds(row0 + b * 28, 28)], dv, sem).wait()

    def quad(sv, dv, J, fire_sv, fire_base):
        for t in range(4):
            pltpu.make_async_copy(gref.at[sv.at[J + t]], rbufs[t], gs[t]).wait()
            pltpu.async_copy(rbufs[t], acc.at[dv.at[J + t]], ss[t], add=True)
        for t in range(4):
            pltpu.make_async_copy(rbufs[t], acc.at[dv.at[J + t]], ss[t]).wait()
            if fire_sv is not None:
                pltpu.async_copy(gref.at[fire_sv.at[fire_base + t]], rbufs[t], gs[t])

    def main_quads(sv, dv):
        def body(k, _):
            quad(sv, dv, 4 * k, sv, 4 * k + 4)
            return 0
        lax.fori_loop(0, 6, body, 0)

    idx_copy(0, srcA, dstA, iA)
    idx_wait(0, srcA, dstA, iA)
    for t in range(4):
        pltpu.async_copy(gref.at[srcA.at[t]], rbufs[t], gs[t])

    nlast = nbulks // 2 - 1

    def bulk_pair(bp, _):
        idx_copy(2 * bp + 1, srcB, dstB, iB)
        main_quads(srcA, dstA)
        idx_wait(2 * bp + 1, srcB, dstB, iB)
        quad(srcA, dstA, 24, srcB, 0)

        @pl.when(bp < nlast)
        def _():
            idx_copy(2 * bp + 2, srcA, dstA, iA)

        main_quads(srcB, dstB)

        @pl.when(bp < nlast)
        def _():
            idx_wait(2 * bp + 2, srcA, dstA, iA)
            quad(srcB, dstB, 24, srcA, 0)

        @pl.when(bp == nlast)
        def _():
            quad(srcB, dstB, 24, None, 0)

        return 0

    lax.fori_loop(0, nbulks // 2, bulk_pair, 0)


def _spmm_scratch(w):
    return [
        pltpu.VMEM((28, 64), jnp.int32),
        pltpu.VMEM((28, 64), jnp.int32),
        pltpu.VMEM((28, 64), jnp.int32),
        pltpu.VMEM((28, 64), jnp.int32),
        pltpu.VMEM((64, w), jnp.float32),
        pltpu.VMEM((64, w), jnp.float32),
        pltpu.VMEM((64, w), jnp.float32),
        pltpu.VMEM((64, w), jnp.float32),
        pltpu.VMEM_SHARED((NP, w), jnp.float32),
    ] + [pltpu.SemaphoreType.DMA] * 10


def _spmm_body(nchunks, args):
    gs = args[:nchunks]
    src2d, dst2d, zrows = args[nchunks:nchunks + 3]
    nouts = (nchunks + 1) // 2
    outs = args[nchunks + 3:nchunks + 3 + nouts]
    sc = args[nchunks + 3 + nouts:]
    acc = sc[8]
    scratch = sc[:8] + sc[9:]
    c = lax.axis_index("c")
    s = lax.axis_index("s")
    _zero_acc(zrows, acc, s)
    plsc.subcore_barrier()

    phases = []
    k = 0
    while k + 2 <= nchunks:
        phases.append((gs[k], gs[k + 1]))
        k += 2
    if k < nchunks:
        phases.append((gs[k],))

    for pi, ph in enumerate(phases):
        if len(ph) == 2:
            @pl.when(c == 0)
            def _(_ph=ph):
                _edge_loop(_ph[0], src2d, dst2d, scratch, acc, s * 784, 28)

            @pl.when(c == 1)
            def _(_ph=ph):
                _edge_loop(_ph[1], src2d, dst2d, scratch, acc, s * 784, 28)
        else:
            _edge_loop(ph[0], src2d, dst2d, scratch, acc,
                       (c * 16 + s) * 392, 14)
        plsc.subcore_barrier()
        pltpu.sync_copy(acc.at[pl.ds(s * 3128, 3128)],
                        outs[pi].at[pl.ds(c * NP + s * 3128, 3128)])
        if pi + 1 < len(phases):
            _zero_acc(zrows, acc, s)
            plsc.subcore_barrier()


def _make_spmm(nchunks, w=CW):
    nouts = (nchunks + 1) // 2

    @functools.partial(
        pl.kernel,
        out_type=[_f32(2 * NP, w) for _ in range(nouts)],
        mesh=_mesh,
        compiler_params=_sc_params,
        scratch_types=_spmm_scratch(w),
    )
    def k(*args):
        _spmm_body(nchunks, args)

    return k


_spmm2 = _make_spmm(2)
_spmm1 = _make_spmm(1)
_spmm1_16 = _make_spmm(1, 16)


BN = 3128
_GRID = (16,)


def _rows_spec(width, half=None):
    if half is None:
        return pl.BlockSpec((BN, width), lambda i: (i, 0))
    off = half * 16
    return pl.BlockSpec((BN, width), lambda i, _o=off: (i + _o, 0))


def _full_spec(shape):
    nd = len(shape)
    return pl.BlockSpec(shape, lambda *_: (0,) * nd)


def _chunk_plan(width):
    ws = []
    rem = width
    while rem > 16:
        ws.append(CW)
        rem -= CW
    if rem > 0:
        ws.append(16)
    while sum(ws) - width >= 16:
        ws.pop()
        ws.append(16)
    return ws


def _prep_body(x_ref, dega_ref, degb_ref, dinv_ref, *g_refs):
    deg = dega_ref[:, :1] + degb_ref[:, :1] + 1.0
    dinv = lax.rsqrt(deg)
    dinv_ref[...] = dinv
    g = x_ref[...] * dinv
    _write_chunks(g, 78, _chunk_plan(78), g_refs)


def _write_chunks(gn, width, widths, outs):
    lo = 0
    for w, oref in zip(widths, outs):
        hi = min(lo + w, width)
        blk = gn[:, lo:hi]
        if hi - lo < w:
            blk = jnp.concatenate(
                [blk, jnp.zeros((BN, w - (hi - lo)), jnp.float32)], axis=1)
        oref[...] = blk
        lo += w


def _dense_body(pieces, width_in, width_out, last, *refs):
    nr = sum(len(p) for p in pieces)
    nci = len(pieces)
    rrefs = refs[:nr]
    gs = refs[nr:nr + nci]
    dinv_ref, w_ref, b_ref = refs[nr + nci:nr + nci + 3]
    outs = refs[nr + nci + 3 + (1 if last else 0):]
    cols = []
    for p in pieces:
        acc = rrefs[p[0]][...]
        for q in p[1:]:
            acc = acc + rrefs[q][...]
        cols.append(acc)
    r = jnp.concatenate(cols, axis=1)[:, :width_in]
    g = jnp.concatenate([ref[...] for ref in gs], axis=1)[:, :width_in]
    dinv = dinv_ref[...]
    ax = dinv * (r + g)
    h = jnp.maximum(jnp.dot(ax, w_ref[...],
                            preferred_element_type=jnp.float32) + b_ref[...], 0.0)
    if last:
        batch_ref = refs[nr + nci + 3]
        h3 = jnp.concatenate(
            [h, jnp.ones((BN, 1), jnp.float32),
             jnp.zeros((BN, 320 - width_out - 1), jnp.float32)], axis=1)
        onehot = (batch_ref[...] ==
                  lax.broadcasted_iota(jnp.int32, (1, 1024), 1)
                  ).astype(jnp.float32)
        contrib = lax.dot_general(
            onehot, h3, (((0,), (0,)), ((), ())),
            preferred_element_type=jnp.float32)

        @pl.when(pl.program_id(0) == 0)
        def _():
            outs[0][...] = jnp.zeros((1024, 320), jnp.float32)

        outs[0][...] += contrib
    else:
        _write_chunks(dinv * h, width_out, _chunk_plan(width_out), outs)


def _dense_layer(r_parts, g_chunks, dinv, W, b, width_in, width_out,
                 last=False, batch_pad=None):
    nci = len(g_chunks)
    assert len(r_parts) == nci
    r_args, r_specs, pieces, idx = [], [], [], 0
    for arr, halves in r_parts:
        plist = []
        for h in halves:
            r_args.append(arr)
            r_specs.append(_rows_spec(arr.shape[1], half=h))
            plist.append(idx)
            idx += 1
        pieces.append(plist)
    if last:
        out_shape = [_f32(1024, 320)]
        out_specs = [_full_spec((1024, 320))]
    else:
        plan = _chunk_plan(width_out)
        out_shape = [_f32(NP, w) for w in plan]
        out_specs = [_rows_spec(w) for w in plan]
    in_specs = (
        r_specs
        + [_rows_spec(g.shape[1]) for g in g_chunks]
        + [_rows_spec(1), _full_spec(W.shape), _full_spec((1, width_out))]
        + ([_rows_spec(1)] if last else [])
    )
    body = functools.partial(_dense_body, pieces, width_in, width_out, last)
    args = r_args + list(g_chunks) + [dinv, W, b.reshape(1, -1)]
    if last:
        args.append(batch_pad)
    outs = pl.pallas_call(
        body, grid=_GRID, in_specs=in_specs, out_specs=out_specs,
        out_shape=out_shape,
    )(*args)
    return outs


def _head_body(pool_ref, gemb_ref, wp_ref, bp_ref, wg_ref, bg_ref,
               wf1_ref, bf1_ref, wf2_ref, bf2_ref, o_ref):
    sums = pool_ref[...]
    counts = sums[:, 312:313]
    x = sums[:, :312] / jnp.maximum(counts, 1.0)
    ge = jnp.dot(gemb_ref[...], wp_ref[...],
                 preferred_element_type=jnp.float32) + bp_ref[...]
    wg = wg_ref[...]
    gate = jax.nn.sigmoid(
        jnp.dot(x, wg[:312], preferred_element_type=jnp.float32)
        + jnp.dot(ge, wg[312:], preferred_element_type=jnp.float32)
        + bg_ref[...])
    fused = gate * ge + (1.0 - gate) * x
    h = jnp.maximum(jnp.dot(fused, wf1_ref[...],
                            preferred_element_type=jnp.float32) + bf1_ref[...], 0.0)
    o_ref[...] = jnp.dot(h, wf2_ref[...],
                         preferred_element_type=jnp.float32) + bf2_ref[...]


def _spmm_all(g_chunks, src2d, dst2d, zrows, zrows16):
    nc = len(g_chunks)
    parts = []
    k = 0
    while k + 2 <= nc:
        (r,) = _spmm2(g_chunks[k], g_chunks[k + 1], src2d, dst2d, zrows)
        parts.append((r, [0]))
        parts.append((r, [1]))
        k += 2
    if k < nc:
        if g_chunks[k].shape[1] == 16:
            (r,) = _spmm1_16(g_chunks[k], src2d, dst2d, zrows16)
        else:
            (r,) = _spmm1(g_chunks[k], src2d, dst2d, zrows)
        parts.append((r, [0, 1]))
    return parts


def kernel(mol_x, mol_edge_index, mol_batch, global_emb, W1, b1, W2, b2,
           W3, b3, Wp, bp, Wg, bg, Wf1, bf1, Wf2, bf2):
    src = mol_edge_index[0].astype(jnp.int32)
    dst = mol_edge_index[1].astype(jnp.int32)
    batch = mol_batch.astype(jnp.int32)

    pad_e = EP - E
    src2d = jnp.concatenate([src, jnp.full((pad_e,), N, jnp.int32)]).reshape(EROWS, 64)
    dst2d = jnp.concatenate([dst, jnp.full((pad_e,), N, jnp.int32)]).reshape(EROWS, 64)
    batch_pad = jnp.concatenate(
        [batch, jnp.full((NP - N,), 1024, jnp.int32)]).reshape(NP, 1)
    x_pad = jnp.concatenate([mol_x, jnp.zeros((NP - N, 78), jnp.float32)])
    ones64 = jnp.ones((64, DW), jnp.float32)
    zcol = jnp.zeros((NP, DW), jnp.float32)
    zrows = jnp.zeros((NP, CW), jnp.float32)
    zrows16 = jnp.zeros((NP, 16), jnp.float32)

    degp = _deg_kernel(dst2d, ones64, zcol)
    prep = pl.pallas_call(
        _prep_body, grid=_GRID,
        in_specs=[_rows_spec(78), _rows_spec(DW, half=0), _rows_spec(DW, half=1)],
        out_specs=[_rows_spec(1)] + [_rows_spec(w) for w in _chunk_plan(78)],
        out_shape=[_f32(NP, 1)] + [_f32(NP, w) for w in _chunk_plan(78)],
    )(x_pad, degp, degp)
    dinv, g1 = prep[0], prep[1:]

    r1 = _spmm_all(g1, src2d, dst2d, zrows, zrows16)
    g2 = _dense_layer(r1, g1, dinv, W1, b1, 78, 78)

    r2 = _spmm_all(g2, src2d, dst2d, zrows, zrows16)
    g3 = _dense_layer(r2, g2, dinv, W2, b2, 78, 156)

    r3 = _spmm_all(g3, src2d, dst2d, zrows, zrows16)
    (pool,) = _dense_layer(r3, g3, dinv, W3, b3, 156, 312, last=True,
                           batch_pad=batch_pad)

    out = pl.pallas_call(
        _head_body,
        in_specs=[_full_spec((1024, 320)), _full_spec((1024, 128)),
                  _full_spec((128, 312)), _full_spec((1, 312)),
                  _full_spec((624, 1)), _full_spec((1, 1)),
                  _full_spec((312, 1024)), _full_spec((1, 1024)),
                  _full_spec((1024, 128)), _full_spec((1, 128))],
        out_specs=_full_spec((1024, 128)),
        out_shape=_f32(1024, 128),
    )(pool, global_emb, Wp, bp.reshape(1, -1), Wg, bg.reshape(1, -1),
      Wf1, bf1.reshape(1, -1), Wf2, bf2.reshape(1, -1))
    return out


# speedup vs baseline: 1.3418x; 1.1478x over previous
import functools

import jax
import jax.numpy as jnp
from jax import lax
from jax.experimental import pallas as pl
from jax.experimental.pallas import tpu as pltpu
from jax.experimental.pallas import tpu_sc as plsc

N = 50000
NP = 50048
E = 800000
EP = 802816
EROWS = EP // 64
GP = 1152
CW = 32

_mesh = plsc.VectorSubcoreMesh(core_axis_name="c", subcore_axis_name="s")
_sc_params = pltpu.CompilerParams(use_tc_tiling_on_sc=False)


def _f32(*shape):
    return jax.ShapeDtypeStruct(shape, jnp.float32)


DW = 16


@functools.partial(
    pl.kernel,
    out_type=_f32(2 * NP, DW),
    mesh=_mesh,
    compiler_params=_sc_params,
    scratch_types=[
        pltpu.VMEM((28, 64), jnp.int32),
        pltpu.VMEM((64, DW), jnp.float32),
        pltpu.VMEM_SHARED((NP, DW), jnp.float32),
    ] + [pltpu.SemaphoreType.DMA] * 4,
)
def _deg_kernel(dst2d, ones64, zcol, deg_out, dstv, onesv, acc, *ss):
    c = lax.axis_index("c")
    s = lax.axis_index("s")
    pltpu.sync_copy(zcol.at[pl.ds(s * 3128, 3128)], acc.at[pl.ds(s * 3128, 3128)])
    pltpu.sync_copy(ones64, onesv)
    plsc.subcore_barrier()
    row0 = (c * 16 + s) * 392

    def bulk(b, _):
        pltpu.sync_copy(dst2d.at[pl.ds(row0 + b * 28, 28)], dstv)

        def quad(q, _):
            for t in range(4):
                pltpu.async_copy(onesv, acc.at[dstv.at[4 * q + t]], ss[t], add=True)
            for t in range(4):
                pltpu.make_async_copy(onesv, acc.at[dstv.at[4 * q + t]], ss[t]).wait()
            return 0

        lax.fori_loop(0, 7, quad, 0)
        return 0

    lax.fori_loop(0, 14, bulk, 0)
    plsc.subcore_barrier()
    pltpu.sync_copy(acc.at[pl.ds(s * 3128, 3128)],
                    deg_out.at[pl.ds(c * NP + s * 3128, 3128)])


def _zero_acc(zrows, acc, s):
    pltpu.sync_copy(zrows.at[pl.ds(s * 3128, 3128)], acc.at[pl.ds(s * 3128, 3128)])


def _edge_loop(gref, src2d, dst2d, sc, acc, row0, nbulks):
    (srcA, dstA, srcB, dstB) = sc[:4]
    rbufs = sc[4:11]
    gs = sc[11:18]
    ss = sc[18:25]
    iA, iB = sc[25:27]

    def idx_copy(b, sv, dv, sem):
        pltpu.async_copy(src2d.at[pl.ds(row0 + b * 28, 28)], sv, sem)
        pltpu.async_copy(dst2d.at[pl.ds(row0 + b * 28, 28)], dv, sem)

    def idx_wait(b, sv, dv, sem):
        pltpu.make_async_copy(src2d.at[pl.ds(row0 + b * 28, 28)], sv, sem).wait()
        pltpu.make_async_copy(dst2d.at[pl.ds(row0 + b * 28, 28)], dv, sem).wait()

    def sept(sv, dv, J, fire_sv, fire_base):
        for t in range(7):
            pltpu.make_async_copy(gref.at[sv.at[J + t]], rbufs[t], gs[t]).wait()
            pltpu.async_copy(rbufs[t], acc.at[dv.at[J + t]], ss[t], add=True)
        for t in range(7):
            pltpu.make_async_copy(rbufs[t], acc.at[dv.at[J + t]], ss[t]).wait()
            if fire_sv is not None:
                pltpu.async_copy(gref.at[fire_sv.at[fire_base + t]], rbufs[t], gs[t])

    def main_septs(sv, dv):
        def body(k, _):
            sept(sv, dv, 7 * k, sv, 7 * k + 7)
            return 0
        lax.fori_loop(0, 3, body, 0)

    idx_copy(0, srcA, dstA, iA)
    idx_wait(0, srcA, dstA, iA)
    for t in range(7):
        pltpu.async_copy(gref.at[srcA.at[t]], rbufs[t], gs[t])

    nlast = nbulks // 2 - 1

    def bulk_pair(bp, _):
        idx_copy(2 * bp + 1, srcB, dstB, iB)
        main_septs(srcA, dstA)
        idx_wait(2 * bp + 1, srcB, dstB, iB)
        sept(srcA, dstA, 21, srcB, 0)

        @pl.when(bp < nlast)
        def _():
            idx_copy(2 * bp + 2, srcA, dstA, iA)

        main_septs(srcB, dstB)

        @pl.when(bp < nlast)
        def _():
            idx_wait(2 * bp + 2, srcA, dstA, iA)
            sept(srcB, dstB, 21, srcA, 0)

        @pl.when(bp == nlast)
        def _():
            sept(srcB, dstB, 21, None, 0)

        return 0

    lax.fori_loop(0, nbulks // 2, bulk_pair, 0)


def _spmm_scratch(w):
    return [
        pltpu.VMEM((28, 64), jnp.int32),
        pltpu.VMEM((28, 64), jnp.int32),
        pltpu.VMEM((28, 64), jnp.int32),
        pltpu.VMEM((28, 64), jnp.int32),
    ] + [pltpu.VMEM((64, w), jnp.float32)] * 7 + [
        pltpu.VMEM_SHARED((NP, w), jnp.float32),
    ] + [pltpu.SemaphoreType.DMA] * 16


def _spmm_body(nchunks, args):
    gs = args[:nchunks]
    src2d, dst2d, zrows = args[nchunks:nchunks + 3]
    nouts = (nchunks + 1) // 2
    outs = args[nchunks + 3:nchunks + 3 + nouts]
    sc = args[nchunks + 3 + nouts:]
    acc = sc[11]
    scratch = sc[:11] + sc[12:]
    c = lax.axis_index("c")
    s = lax.axis_index("s")
    _zero_acc(zrows, acc, s)
    plsc.subcore_barrier()

    phases = []
    k = 0
    while k + 2 <= nchunks:
        phases.append((gs[k], gs[k + 1]))
        k += 2
    if k < nchunks:
        phases.append((gs[k],))

    for pi, ph in enumerate(phases):
        if len(ph) == 2:
            @pl.when(c == 0)
            def _(_ph=ph):
                _edge_loop(_ph[0], src2d, dst2d, scratch, acc, s * 784, 28)

            @pl.when(c == 1)
            def _(_ph=ph):
                _edge_loop(_ph[1], src2d, dst2d, scratch, acc, s * 784, 28)
        else:
            _edge_loop(ph[0], src2d, dst2d, scratch, acc,
                       (c * 16 + s) * 392, 14)
        plsc.subcore_barrier()
        pltpu.sync_copy(acc.at[pl.ds(s * 3128, 3128)],
                        outs[pi].at[pl.ds(c * NP + s * 3128, 3128)])
        if pi + 1 < len(phases):
            _zero_acc(zrows, acc, s)
            plsc.subcore_barrier()


def _make_spmm(nchunks, w=CW):
    nouts = (nchunks + 1) // 2

    @functools.partial(
        pl.kernel,
        out_type=[_f32(2 * NP, w) for _ in range(nouts)],
        mesh=_mesh,
        compiler_params=_sc_params,
        scratch_types=_spmm_scratch(w),
    )
    def k(*args):
        _spmm_body(nchunks, args)

    return k


_spmm2 = _make_spmm(2)
_spmm1 = _make_spmm(1)
_spmm1_16 = _make_spmm(1, 16)


BN = 3128
_GRID = (16,)


def _rows_spec(width, half=None):
    if half is None:
        return pl.BlockSpec((BN, width), lambda i: (i, 0))
    off = half * 16
    return pl.BlockSpec((BN, width), lambda i, _o=off: (i + _o, 0))


def _full_spec(shape):
    nd = len(shape)
    return pl.BlockSpec(shape, lambda *_: (0,) * nd)


def _chunk_plan(width):
    ws = []
    rem = width
    while rem > 16:
        ws.append(CW)
        rem -= CW
    if rem > 0:
        ws.append(16)
    while sum(ws) - width >= 16:
        ws.pop()
        ws.append(16)
    return ws


def _prep_body(x_ref, dega_ref, degb_ref, dinv_ref, *g_refs):
    deg = dega_ref[:, :1] + degb_ref[:, :1] + 1.0
    dinv = lax.rsqrt(deg)
    dinv_ref[...] = dinv
    g = x_ref[...] * dinv
    _write_chunks(g, 78, _chunk_plan(78), g_refs)


def _write_chunks(gn, width, widths, outs):
    lo = 0
    for w, oref in zip(widths, outs):
        hi = min(lo + w, width)
        blk = gn[:, lo:hi]
        if hi - lo < w:
            blk = jnp.concatenate(
                [blk, jnp.zeros((BN, w - (hi - lo)), jnp.float32)], axis=1)
        oref[...] = blk
        lo += w


def _dense_body(pieces, width_in, width_out, last, *refs):
    nr = sum(len(p) for p in pieces)
    nci = len(pieces)
    rrefs = refs[:nr]
    gs = refs[nr:nr + nci]
    dinv_ref, w_ref, b_ref = refs[nr + nci:nr + nci + 3]
    outs = refs[nr + nci + 3 + (1 if last else 0):]
    cols = []
    for p in pieces:
        acc = rrefs[p[0]][...]
        for q in p[1:]:
            acc = acc + rrefs[q][...]
        cols.append(acc)
    r = jnp.concatenate(cols, axis=1)[:, :width_in]
    g = jnp.concatenate([ref[...] for ref in gs], axis=1)[:, :width_in]
    dinv = dinv_ref[...]
    ax = dinv * (r + g)
    h = jnp.maximum(jnp.dot(ax, w_ref[...],
                            preferred_element_type=jnp.float32) + b_ref[...], 0.0)
    if last:
        batch_ref = refs[nr + nci + 3]
        h3 = jnp.concatenate(
            [h, jnp.ones((BN, 1), jnp.float32),
             jnp.zeros((BN, 320 - width_out - 1), jnp.float32)], axis=1)
        onehot = (batch_ref[...] ==
                  lax.broadcasted_iota(jnp.int32, (1, 1024), 1)
                  ).astype(jnp.float32)
        contrib = lax.dot_general(
            onehot, h3, (((0,), (0,)), ((), ())),
            preferred_element_type=jnp.float32)

        @pl.when(pl.program_id(0) == 0)
        def _():
            outs[0][...] = jnp.zeros((1024, 320), jnp.float32)

        outs[0][...] += contrib
    else:
        _write_chunks(dinv * h, width_out, _chunk_plan(width_out), outs)


def _dense_layer(r_parts, g_chunks, dinv, W, b, width_in, width_out,
                 last=False, batch_pad=None):
    nci = len(g_chunks)
    assert len(r_parts) == nci
    r_args, r_specs, pieces, idx = [], [], [], 0
    for arr, halves in r_parts:
        plist = []
        for h in halves:
            r_args.append(arr)
            r_specs.append(_rows_spec(arr.shape[1], half=h))
            plist.append(idx)
            idx += 1
        pieces.append(plist)
    if last:
        out_shape = [_f32(1024, 320)]
        out_specs = [_full_spec((1024, 320))]
    else:
        plan = _chunk_plan(width_out)
        out_shape = [_f32(NP, w) for w in plan]
        out_specs = [_rows_spec(w) for w in plan]
    in_specs = (
        r_specs
        + [_rows_spec(g.shape[1]) for g in g_chunks]
        + [_rows_spec(1), _full_spec(W.shape), _full_spec((1, width_out))]
        + ([_rows_spec(1)] if last else [])
    )
    body = functools.partial(_dense_body, pieces, width_in, width_out, last)
    args = r_args + list(g_chunks) + [dinv, W, b.reshape(1, -1)]
    if last:
        args.append(batch_pad)
    outs = pl.pallas_call(
        body, grid=_GRID, in_specs=in_specs, out_specs=out_specs,
        out_shape=out_shape,
    )(*args)
    return outs


def _head_body(pool_ref, gemb_ref, wp_ref, bp_ref, wg_ref, bg_ref,
               wf1_ref, bf1_ref, wf2_ref, bf2_ref, o_ref):
    sums = pool_ref[...]
    counts = sums[:, 312:313]
    x = sums[:, :312] / jnp.maximum(counts, 1.0)
    ge = jnp.dot(gemb_ref[...], wp_ref[...],
                 preferred_element_type=jnp.float32) + bp_ref[...]
    wg = wg_ref[...]
    gate = jax.nn.sigmoid(
        jnp.dot(x, wg[:312], preferred_element_type=jnp.float32)
        + jnp.dot(ge, wg[312:], preferred_element_type=jnp.float32)
        + bg_ref[...])
    fused = gate * ge + (1.0 - gate) * x
    h = jnp.maximum(jnp.dot(fused, wf1_ref[...],
                            preferred_element_type=jnp.float32) + bf1_ref[...], 0.0)
    o_ref[...] = jnp.dot(h, wf2_ref[...],
                         preferred_element_type=jnp.float32) + bf2_ref[...]


def _spmm_all(g_chunks, src2d, dst2d, zrows, zrows16):
    nc = len(g_chunks)
    parts = []
    k = 0
    while k + 2 <= nc:
        (r,) = _spmm2(g_chunks[k], g_chunks[k + 1], src2d, dst2d, zrows)
        parts.append((r, [0]))
        parts.append((r, [1]))
        k += 2
    if k < nc:
        if g_chunks[k].shape[1] == 16:
            (r,) = _spmm1_16(g_chunks[k], src2d, dst2d, zrows16)
        else:
            (r,) = _spmm1(g_chunks[k], src2d, dst2d, zrows)
        parts.append((r, [0, 1]))
    return parts


def kernel(mol_x, mol_edge_index, mol_batch, global_emb, W1, b1, W2, b2,
           W3, b3, Wp, bp, Wg, bg, Wf1, bf1, Wf2, bf2):
    src = mol_edge_index[0].astype(jnp.int32)
    dst = mol_edge_index[1].astype(jnp.int32)
    batch = mol_batch.astype(jnp.int32)

    pad_e = EP - E
    src2d = jnp.concatenate([src, jnp.full((pad_e,), N, jnp.int32)]).reshape(EROWS, 64)
    dst2d = jnp.concatenate([dst, jnp.full((pad_e,), N, jnp.int32)]).reshape(EROWS, 64)
    batch_pad = jnp.concatenate(
        [batch, jnp.full((NP - N,), 1024, jnp.int32)]).reshape(NP, 1)
    x_pad = jnp.concatenate([mol_x, jnp.zeros((NP - N, 78), jnp.float32)])
    ones64 = jnp.ones((64, DW), jnp.float32)
    zcol = jnp.zeros((NP, DW), jnp.float32)
    zrows = jnp.zeros((NP, CW), jnp.float32)
    zrows16 = jnp.zeros((NP, 16), jnp.float32)

    degp = _deg_kernel(dst2d, ones64, zcol)
    prep = pl.pallas_call(
        _prep_body, grid=_GRID,
        in_specs=[_rows_spec(78), _rows_spec(DW, half=0), _rows_spec(DW, half=1)],
        out_specs=[_rows_spec(1)] + [_rows_spec(w) for w in _chunk_plan(78)],
        out_shape=[_f32(NP, 1)] + [_f32(NP, w) for w in _chunk_plan(78)],
    )(x_pad, degp, degp)
    dinv, g1 = prep[0], prep[1:]

    r1 = _spmm_all(g1, src2d, dst2d, zrows, zrows16)
    g2 = _dense_layer(r1, g1, dinv, W1, b1, 78, 78)

    r2 = _spmm_all(g2, src2d, dst2d, zrows, zrows16)
    g3 = _dense_layer(r2, g2, dinv, W2, b2, 78, 156)

    r3 = _spmm_all(g3, src2d, dst2d, zrows, zrows16)
    (pool,) = _dense_layer(r3, g3, dinv, W3, b3, 156, 312, last=True,
                           batch_pad=batch_pad)

    out = pl.pallas_call(
        _head_body,
        in_specs=[_full_spec((1024, 320)), _full_spec((1024, 128)),
                  _full_spec((128, 312)), _full_spec((1, 312)),
                  _full_spec((624, 1)), _full_spec((1, 1)),
                  _full_spec((312, 1024)), _full_spec((1, 1024)),
                  _full_spec((1024, 128)), _full_spec((1, 128))],
        out_specs=_full_spec((1024, 128)),
        out_shape=_f32(1024, 128),
    )(pool, global_emb, Wp, bp.reshape(1, -1), Wg, bg.reshape(1, -1),
      Wf1, bf1.reshape(1, -1), Wf2, bf2.reshape(1, -1))
    return out


# speedup vs baseline: 1.3454x; 1.0027x over previous
import functools

import jax
import jax.numpy as jnp
from jax import lax
from jax.experimental import pallas as pl
from jax.experimental.pallas import tpu as pltpu
from jax.experimental.pallas import tpu_sc as plsc

N = 50000
NP = 50048
E = 800000
EP = 802816
EROWS = EP // 64
GP = 1152
CW = 32

_mesh = plsc.VectorSubcoreMesh(core_axis_name="c", subcore_axis_name="s")
_sc_params = pltpu.CompilerParams(use_tc_tiling_on_sc=False)


def _f32(*shape):
    return jax.ShapeDtypeStruct(shape, jnp.float32)


DW = 16


@functools.partial(
    pl.kernel,
    out_type=_f32(2 * NP, DW),
    mesh=_mesh,
    compiler_params=_sc_params,
    scratch_types=[
        pltpu.VMEM((28, 64), jnp.int32),
        pltpu.VMEM((64, DW), jnp.float32),
        pltpu.VMEM_SHARED((NP, DW), jnp.float32),
    ] + [pltpu.SemaphoreType.DMA] * 4,
)
def _deg_kernel(dst2d, ones64, zcol, deg_out, dstv, onesv, acc, *ss):
    c = lax.axis_index("c")
    s = lax.axis_index("s")
    pltpu.sync_copy(zcol.at[pl.ds(s * 3128, 3128)], acc.at[pl.ds(s * 3128, 3128)])
    pltpu.sync_copy(ones64, onesv)
    plsc.subcore_barrier()
    row0 = (c * 16 + s) * 392

    def bulk(b, _):
        pltpu.sync_copy(dst2d.at[pl.ds(row0 + b * 28, 28)], dstv)

        def fire(q, _):
            for t in range(4):
                pltpu.async_copy(onesv, acc.at[dstv.at[4 * q + t]], ss[t], add=True)
            return 0

        lax.fori_loop(0, 7, fire, 0)

        def drain(q, _):
            for t in range(4):
                pltpu.make_async_copy(onesv, acc.at[dstv.at[4 * q + t]], ss[t]).wait()
            return 0

        lax.fori_loop(0, 7, drain, 0)
        return 0

    lax.fori_loop(0, 14, bulk, 0)
    plsc.subcore_barrier()
    pltpu.sync_copy(acc.at[pl.ds(s * 3128, 3128)],
                    deg_out.at[pl.ds(c * NP + s * 3128, 3128)])


def _zero_acc(zrows, acc, s):
    pltpu.sync_copy(zrows.at[pl.ds(s * 3128, 3128)], acc.at[pl.ds(s * 3128, 3128)])


def _edge_loop(gref, src2d, dst2d, sc, acc, row0, nbulks):
    (srcA, dstA, srcB, dstB) = sc[:4]
    rbufs = sc[4:11]
    gs = sc[11:18]
    ss = sc[18:25]
    iA, iB = sc[25:27]

    def idx_copy(b, sv, dv, sem):
        pltpu.async_copy(src2d.at[pl.ds(row0 + b * 28, 28)], sv, sem)
        pltpu.async_copy(dst2d.at[pl.ds(row0 + b * 28, 28)], dv, sem)

    def idx_wait(b, sv, dv, sem):
        pltpu.make_async_copy(src2d.at[pl.ds(row0 + b * 28, 28)], sv, sem).wait()
        pltpu.make_async_copy(dst2d.at[pl.ds(row0 + b * 28, 28)], dv, sem).wait()

    def sept(sv, dv, J, fire_sv, fire_base):
        for t in range(7):
            pltpu.make_async_copy(gref.at[sv.at[J + t]], rbufs[t], gs[t]).wait()
            pltpu.async_copy(rbufs[t], acc.at[dv.at[J + t]], ss[t], add=True)
        for t in range(7):
            pltpu.make_async_copy(rbufs[t], acc.at[dv.at[J + t]], ss[t]).wait()
            if fire_sv is not None:
                pltpu.async_copy(gref.at[fire_sv.at[fire_base + t]], rbufs[t], gs[t])

    def main_septs(sv, dv):
        def body(k, _):
            sept(sv, dv, 7 * k, sv, 7 * k + 7)
            return 0
        lax.fori_loop(0, 3, body, 0)

    idx_copy(0, srcA, dstA, iA)
    idx_wait(0, srcA, dstA, iA)
    for t in range(7):
        pltpu.async_copy(gref.at[srcA.at[t]], rbufs[t], gs[t])

    nlast = nbulks // 2 - 1

    def bulk_pair(bp, _):
        idx_copy(2 * bp + 1, srcB, dstB, iB)
        main_septs(srcA, dstA)
        idx_wait(2 * bp + 1, srcB, dstB, iB)
        sept(srcA, dstA, 21, srcB, 0)

        @pl.when(bp < nlast)
        def _():
            idx_copy(2 * bp + 2, srcA, dstA, iA)

        main_septs(srcB, dstB)

        @pl.when(bp < nlast)
        def _():
            idx_wait(2 * bp + 2, srcA, dstA, iA)
            sept(srcB, dstB, 21, srcA, 0)

        @pl.when(bp == nlast)
        def _():
            sept(srcB, dstB, 21, None, 0)

        return 0

    lax.fori_loop(0, nbulks // 2, bulk_pair, 0)


def _spmm_scratch(w):
    return [
        pltpu.VMEM((28, 64), jnp.int32),
        pltpu.VMEM((28, 64), jnp.int32),
        pltpu.VMEM((28, 64), jnp.int32),
        pltpu.VMEM((28, 64), jnp.int32),
    ] + [pltpu.VMEM((64, w), jnp.float32)] * 7 + [
        pltpu.VMEM_SHARED((NP, w), jnp.float32),
    ] + [pltpu.SemaphoreType.DMA] * 16


def _spmm_body(nchunks, args):
    gs = args[:nchunks]
    src2d, dst2d, zrows = args[nchunks:nchunks + 3]
    nouts = (nchunks + 1) // 2
    outs = args[nchunks + 3:nchunks + 3 + nouts]
    sc = args[nchunks + 3 + nouts:]
    acc = sc[11]
    scratch = sc[:11] + sc[12:]
    c = lax.axis_index("c")
    s = lax.axis_index("s")
    _zero_acc(zrows, acc, s)
    plsc.subcore_barrier()

    phases = []
    k = 0
    while k + 2 <= nchunks:
        phases.append((gs[k], gs[k + 1]))
        k += 2
    if k < nchunks:
        phases.append((gs[k],))

    for pi, ph in enumerate(phases):
        if len(ph) == 2:
            @pl.when(c == 0)
            def _(_ph=ph):
                _edge_loop(_ph[0], src2d, dst2d, scratch, acc, s * 784, 28)

            @pl.when(c == 1)
            def _(_ph=ph):
                _edge_loop(_ph[1], src2d, dst2d, scratch, acc, s * 784, 28)
        else:
            _edge_loop(ph[0], src2d, dst2d, scratch, acc,
                       (c * 16 + s) * 392, 14)
        plsc.subcore_barrier()
        pltpu.sync_copy(acc.at[pl.ds(s * 3128, 3128)],
                        outs[pi].at[pl.ds(c * NP + s * 3128, 3128)])
        if pi + 1 < len(phases):
            _zero_acc(zrows, acc, s)
            plsc.subcore_barrier()


def _make_spmm(nchunks, w=CW):
    nouts = (nchunks + 1) // 2

    @functools.partial(
        pl.kernel,
        out_type=[_f32(2 * NP, w) for _ in range(nouts)],
        mesh=_mesh,
        compiler_params=_sc_params,
        scratch_types=_spmm_scratch(w),
    )
    def k(*args):
        _spmm_body(nchunks, args)

    return k


_spmm2 = _make_spmm(2)
_spmm1 = _make_spmm(1)
_spmm1_16 = _make_spmm(1, 16)


BN = 3128
_GRID = (16,)


def _rows_spec(width, half=None):
    if half is None:
        return pl.BlockSpec((BN, width), lambda i: (i, 0))
    off = half * 16
    return pl.BlockSpec((BN, width), lambda i, _o=off: (i + _o, 0))


def _full_spec(shape):
    nd = len(shape)
    return pl.BlockSpec(shape, lambda *_: (0,) * nd)


def _chunk_plan(width):
    ws = []
    rem = width
    while rem > 16:
        ws.append(CW)
        rem -= CW
    if rem > 0:
        ws.append(16)
    while sum(ws) - width >= 16:
        ws.pop()
        ws.append(16)
    return ws


def _prep_body(x_ref, dega_ref, degb_ref, dinv_ref, *g_refs):
    deg = dega_ref[:, :1] + degb_ref[:, :1] + 1.0
    dinv = lax.rsqrt(deg)
    dinv_ref[...] = dinv
    g = x_ref[...] * dinv
    _write_chunks(g, 78, _chunk_plan(78), g_refs)


def _write_chunks(gn, width, widths, outs):
    lo = 0
    for w, oref in zip(widths, outs):
        hi = min(lo + w, width)
        blk = gn[:, lo:hi]
        if hi - lo < w:
            blk = jnp.concatenate(
                [blk, jnp.zeros((BN, w - (hi - lo)), jnp.float32)], axis=1)
        oref[...] = blk
        lo += w


def _dense_body(pieces, width_in, width_out, last, *refs):
    nr = sum(len(p) for p in pieces)
    nci = len(pieces)
    rrefs = refs[:nr]
    gs = refs[nr:nr + nci]
    dinv_ref, w_ref, b_ref = refs[nr + nci:nr + nci + 3]
    outs = refs[nr + nci + 3 + (1 if last else 0):]
    cols = []
    for p in pieces:
        acc = rrefs[p[0]][...]
        for q in p[1:]:
            acc = acc + rrefs[q][...]
        cols.append(acc)
    r = jnp.concatenate(cols, axis=1)[:, :width_in]
    g = jnp.concatenate([ref[...] for ref in gs], axis=1)[:, :width_in]
    dinv = dinv_ref[...]
    ax = dinv * (r + g)
    h = jnp.maximum(jnp.dot(ax, w_ref[...],
                            preferred_element_type=jnp.float32) + b_ref[...], 0.0)
    if last:
        batch_ref = refs[nr + nci + 3]
        h3 = jnp.concatenate(
            [h, jnp.ones((BN, 1), jnp.float32),
             jnp.zeros((BN, 320 - width_out - 1), jnp.float32)], axis=1)
        onehot = (batch_ref[...] ==
                  lax.broadcasted_iota(jnp.int32, (1, 1024), 1)
                  ).astype(jnp.float32)
        contrib = lax.dot_general(
            onehot, h3, (((0,), (0,)), ((), ())),
            preferred_element_type=jnp.float32)

        @pl.when(pl.program_id(0) == 0)
        def _():
            outs[0][...] = jnp.zeros((1024, 320), jnp.float32)

        outs[0][...] += contrib
    else:
        _write_chunks(dinv * h, width_out, _chunk_plan(width_out), outs)


def _dense_layer(r_parts, g_chunks, dinv, W, b, width_in, width_out,
                 last=False, batch_pad=None):
    nci = len(g_chunks)
    assert len(r_parts) == nci
    r_args, r_specs, pieces, idx = [], [], [], 0
    for arr, halves in r_parts:
        plist = []
        for h in halves:
            r_args.append(arr)
            r_specs.append(_rows_spec(arr.shape[1], half=h))
            plist.append(idx)
            idx += 1
        pieces.append(plist)
    if last:
        out_shape = [_f32(1024, 320)]
        out_specs = [_full_spec((1024, 320))]
    else:
        plan = _chunk_plan(width_out)
        out_shape = [_f32(NP, w) for w in plan]
        out_specs = [_rows_spec(w) for w in plan]
    in_specs = (
        r_specs
        + [_rows_spec(g.shape[1]) for g in g_chunks]
        + [_rows_spec(1), _full_spec(W.shape), _full_spec((1, width_out))]
        + ([_rows_spec(1)] if last else [])
    )
    body = functools.partial(_dense_body, pieces, width_in, width_out, last)
    args = r_args + list(g_chunks) + [dinv, W, b.reshape(1, -1)]
    if last:
        args.append(batch_pad)
    outs = pl.pallas_call(
        body, grid=_GRID, in_specs=in_specs, out_specs=out_specs,
        out_shape=out_shape,
    )(*args)
    return outs


def _head_body(pool_ref, gemb_ref, wp_ref, bp_ref, wg_ref, bg_ref,
               wf1_ref, bf1_ref, wf2_ref, bf2_ref, o_ref):
    sums = pool_ref[...]
    counts = sums[:, 312:313]
    x = sums[:, :312] / jnp.maximum(counts, 1.0)
    ge = jnp.dot(gemb_ref[...], wp_ref[...],
                 preferred_element_type=jnp.float32) + bp_ref[...]
    wg = wg_ref[...]
    gate = jax.nn.sigmoid(
        jnp.dot(x, wg[:312], preferred_element_type=jnp.float32)
        + jnp.dot(ge, wg[312:], preferred_element_type=jnp.float32)
        + bg_ref[...])
    fused = gate * ge + (1.0 - gate) * x
    h = jnp.maximum(jnp.dot(fused, wf1_ref[...],
                            preferred_element_type=jnp.float32) + bf1_ref[...], 0.0)
    o_ref[...] = jnp.dot(h, wf2_ref[...],
                         preferred_element_type=jnp.float32) + bf2_ref[...]


def _spmm_all(g_chunks, src2d, dst2d, zrows, zrows16):
    nc = len(g_chunks)
    parts = []
    k = 0
    while k + 2 <= nc:
        (r,) = _spmm2(g_chunks[k], g_chunks[k + 1], src2d, dst2d, zrows)
        parts.append((r, [0]))
        parts.append((r, [1]))
        k += 2
    if k < nc:
        if g_chunks[k].shape[1] == 16:
            (r,) = _spmm1_16(g_chunks[k], src2d, dst2d, zrows16)
        else:
            (r,) = _spmm1(g_chunks[k], src2d, dst2d, zrows)
        parts.append((r, [0, 1]))
    return parts


def kernel(mol_x, mol_edge_index, mol_batch, global_emb, W1, b1, W2, b2,
           W3, b3, Wp, bp, Wg, bg, Wf1, bf1, Wf2, bf2):
    src = mol_edge_index[0].astype(jnp.int32)
    dst = mol_edge_index[1].astype(jnp.int32)
    batch = mol_batch.astype(jnp.int32)

    pad_e = EP - E
    src2d = jnp.concatenate([src, jnp.full((pad_e,), N, jnp.int32)]).reshape(EROWS, 64)
    dst2d = jnp.concatenate([dst, jnp.full((pad_e,), N, jnp.int32)]).reshape(EROWS, 64)
    batch_pad = jnp.concatenate(
        [batch, jnp.full((NP - N,), 1024, jnp.int32)]).reshape(NP, 1)
    x_pad = jnp.concatenate([mol_x, jnp.zeros((NP - N, 78), jnp.float32)])
    ones64 = jnp.ones((64, DW), jnp.float32)
    zcol = jnp.zeros((NP, DW), jnp.float32)
    zrows = jnp.zeros((NP, CW), jnp.float32)
    zrows16 = jnp.zeros((NP, 16), jnp.float32)

    degp = _deg_kernel(dst2d, ones64, zcol)
    prep = pl.pallas_call(
        _prep_body, grid=_GRID,
        in_specs=[_rows_spec(78), _rows_spec(DW, half=0), _rows_spec(DW, half=1)],
        out_specs=[_rows_spec(1)] + [_rows_spec(w) for w in _chunk_plan(78)],
        out_shape=[_f32(NP, 1)] + [_f32(NP, w) for w in _chunk_plan(78)],
    )(x_pad, degp, degp)
    dinv, g1 = prep[0], prep[1:]

    r1 = _spmm_all(g1, src2d, dst2d, zrows, zrows16)
    g2 = _dense_layer(r1, g1, dinv, W1, b1, 78, 78)

    r2 = _spmm_all(g2, src2d, dst2d, zrows, zrows16)
    g3 = _dense_layer(r2, g2, dinv, W2, b2, 78, 156)

    r3 = _spmm_all(g3, src2d, dst2d, zrows, zrows16)
    (pool,) = _dense_layer(r3, g3, dinv, W3, b3, 156, 312, last=True,
                           batch_pad=batch_pad)

    out = pl.pallas_call(
        _head_body,
        in_specs=[_full_spec((1024, 320)), _full_spec((1024, 128)),
                  _full_spec((128, 312)), _full_spec((1, 312)),
                  _full_spec((624, 1)), _full_spec((1, 1)),
                  _full_spec((312, 1024)), _full_spec((1, 1024)),
                  _full_spec((1024, 128)), _full_spec((1, 128))],
        out_specs=_full_spec((1024, 128)),
        out_shape=_f32(1024, 128),
    )(pool, global_emb, Wp, bp.reshape(1, -1), Wg, bg.reshape(1, -1),
      Wf1, bf1.reshape(1, -1), Wf2, bf2.reshape(1, -1))
    return out


# speedup vs baseline: 1.3654x; 1.0149x over previous
import functools

import jax
import jax.numpy as jnp
from jax import lax
from jax.experimental import pallas as pl
from jax.experimental.pallas import tpu as pltpu
from jax.experimental.pallas import tpu_sc as plsc

N = 50000
NP = 50048
E = 800000
EP = 802816
EROWS = EP // 64
GP = 1152
CW = 32

_mesh = plsc.VectorSubcoreMesh(core_axis_name="c", subcore_axis_name="s")
_sc_params = pltpu.CompilerParams(use_tc_tiling_on_sc=False)


def _f32(*shape):
    return jax.ShapeDtypeStruct(shape, jnp.float32)


DW = 16


@functools.partial(
    pl.kernel,
    out_type=_f32(2 * NP, DW),
    mesh=_mesh,
    compiler_params=_sc_params,
    scratch_types=[
        pltpu.VMEM((28, 64), jnp.int32),
        pltpu.VMEM((64, DW), jnp.float32),
        pltpu.VMEM_SHARED((NP, DW), jnp.float32),
    ] + [pltpu.SemaphoreType.DMA] * 4,
)
def _deg_kernel(dst2d, ones64, zcol, deg_out, dstv, onesv, acc, *ss):
    c = lax.axis_index("c")
    s = lax.axis_index("s")
    pltpu.sync_copy(zcol.at[pl.ds(s * 3128, 3128)], acc.at[pl.ds(s * 3128, 3128)])
    pltpu.sync_copy(ones64, onesv)
    plsc.subcore_barrier()
    row0 = (c * 16 + s) * 392

    def bulk(b, _):
        pltpu.sync_copy(dst2d.at[pl.ds(row0 + b * 28, 28)], dstv)

        def fire(q, _):
            for t in range(4):
                pltpu.async_copy(onesv, acc.at[dstv.at[4 * q + t]], ss[t], add=True)
            return 0

        lax.fori_loop(0, 7, fire, 0)

        def drain(q, _):
            for t in range(4):
                pltpu.make_async_copy(onesv, acc.at[dstv.at[4 * q + t]], ss[t]).wait()
            return 0

        lax.fori_loop(0, 7, drain, 0)
        return 0

    lax.fori_loop(0, 14, bulk, 0)
    plsc.subcore_barrier()
    pltpu.sync_copy(acc.at[pl.ds(s * 3128, 3128)],
                    deg_out.at[pl.ds(c * NP + s * 3128, 3128)])


def _zero_acc(zrows, acc, s):
    pltpu.sync_copy(zrows.at[pl.ds(s * 3128, 3128)], acc.at[pl.ds(s * 3128, 3128)])


def _edge_loop(gref, src2d, dst2d, sc, acc, row0, nbulks):
    (srcA, dstA, srcB, dstB) = sc[:4]
    rbufs = sc[4:11]
    gs = sc[11:18]
    ss = sc[18:25]
    iA, iB = sc[25:27]

    def idx_copy(b, sv, dv, sem):
        pltpu.async_copy(src2d.at[pl.ds(row0 + b * 28, 28)], sv, sem)
        pltpu.async_copy(dst2d.at[pl.ds(row0 + b * 28, 28)], dv, sem)

    def idx_wait(b, sv, dv, sem):
        pltpu.make_async_copy(src2d.at[pl.ds(row0 + b * 28, 28)], sv, sem).wait()
        pltpu.make_async_copy(dst2d.at[pl.ds(row0 + b * 28, 28)], dv, sem).wait()

    def sept(sv, dv, J, fire_sv, fire_base):
        for t in range(7):
            pltpu.make_async_copy(gref.at[sv.at[J + t]], rbufs[t], gs[t]).wait()
            pltpu.async_copy(rbufs[t], acc.at[dv.at[J + t]], ss[t], add=True)
        for t in range(7):
            pltpu.make_async_copy(rbufs[t], acc.at[dv.at[J + t]], ss[t]).wait()
            if fire_sv is not None:
                pltpu.async_copy(gref.at[fire_sv.at[fire_base + t]], rbufs[t], gs[t])

    def main_septs(sv, dv):
        def body(k, _):
            sept(sv, dv, 7 * k, sv, 7 * k + 7)
            return 0
        lax.fori_loop(0, 3, body, 0)

    idx_copy(0, srcA, dstA, iA)
    idx_wait(0, srcA, dstA, iA)
    for t in range(7):
        pltpu.async_copy(gref.at[srcA.at[t]], rbufs[t], gs[t])

    nlast = nbulks // 2 - 1

    def bulk_pair(bp, _):
        idx_copy(2 * bp + 1, srcB, dstB, iB)
        main_septs(srcA, dstA)
        idx_wait(2 * bp + 1, srcB, dstB, iB)
        sept(srcA, dstA, 21, srcB, 0)

        @pl.when(bp < nlast)
        def _():
            idx_copy(2 * bp + 2, srcA, dstA, iA)

        main_septs(srcB, dstB)

        @pl.when(bp < nlast)
        def _():
            idx_wait(2 * bp + 2, srcA, dstA, iA)
            sept(srcB, dstB, 21, srcA, 0)

        @pl.when(bp == nlast)
        def _():
            sept(srcB, dstB, 21, None, 0)

        return 0

    lax.fori_loop(0, nbulks // 2, bulk_pair, 0)


def _spmm_scratch(w):
    return [
        pltpu.VMEM((28, 64), jnp.int32),
        pltpu.VMEM((28, 64), jnp.int32),
        pltpu.VMEM((28, 64), jnp.int32),
        pltpu.VMEM((28, 64), jnp.int32),
    ] + [pltpu.VMEM((64, w), jnp.float32)] * 7 + [
        pltpu.VMEM_SHARED((NP, w), jnp.float32),
    ] + [pltpu.SemaphoreType.DMA] * 16


def _spmm_body(nchunks, args):
    gs = args[:nchunks]
    src2d, dst2d, zrows = args[nchunks:nchunks + 3]
    nouts = (nchunks + 1) // 2
    outs = args[nchunks + 3:nchunks + 3 + nouts]
    sc = args[nchunks + 3 + nouts:]
    acc = sc[11]
    scratch = sc[:11] + sc[12:]
    c = lax.axis_index("c")
    s = lax.axis_index("s")
    _zero_acc(zrows, acc, s)
    plsc.subcore_barrier()

    phases = []
    k = 0
    while k + 2 <= nchunks:
        phases.append((gs[k], gs[k + 1]))
        k += 2
    if k < nchunks:
        phases.append((gs[k],))

    for pi, ph in enumerate(phases):
        if len(ph) == 2:
            @pl.when(c == 0)
            def _(_ph=ph):
                _edge_loop(_ph[0], src2d, dst2d, scratch, acc, s * 784, 28)

            @pl.when(c == 1)
            def _(_ph=ph):
                _edge_loop(_ph[1], src2d, dst2d, scratch, acc, s * 784, 28)
        else:
            _edge_loop(ph[0], src2d, dst2d, scratch, acc,
                       (c * 16 + s) * 392, 14)
        plsc.subcore_barrier()
        pltpu.sync_copy(acc.at[pl.ds(s * 3128, 3128)],
                        outs[pi].at[pl.ds(c * NP + s * 3128, 3128)])
        if pi + 1 < len(phases):
            _zero_acc(zrows, acc, s)
            plsc.subcore_barrier()


def _make_spmm(nchunks, w=CW):
    nouts = (nchunks + 1) // 2

    @functools.partial(
        pl.kernel,
        out_type=[_f32(2 * NP, w) for _ in range(nouts)],
        mesh=_mesh,
        compiler_params=_sc_params,
        scratch_types=_spmm_scratch(w),
    )
    def k(*args):
        _spmm_body(nchunks, args)

    return k


_spmm2 = _make_spmm(2)
_spmm1 = _make_spmm(1)
_spmm1_16 = _make_spmm(1, 16)


BN = 3128
_GRID = (16,)


def _rows_spec(width, half=None):
    if half is None:
        return pl.BlockSpec((BN, width), lambda i: (i, 0))
    off = half * 16
    return pl.BlockSpec((BN, width), lambda i, _o=off: (i + _o, 0))


def _full_spec(shape):
    nd = len(shape)
    return pl.BlockSpec(shape, lambda *_: (0,) * nd)


def _chunk_plan(width):
    ws = []
    rem = width
    while rem > 16:
        ws.append(CW)
        rem -= CW
    if rem > 0:
        ws.append(16)
    while sum(ws) - width >= 16:
        ws.pop()
        ws.append(16)
    return ws


def _prep_body(x_ref, dega_ref, degb_ref, dinv_ref, *g_refs):
    deg = dega_ref[:, :1] + degb_ref[:, :1] + 1.0
    dinv = lax.rsqrt(deg)
    dinv_ref[...] = dinv
    g = x_ref[...] * dinv
    _write_chunks(g, 78, _chunk_plan(78), g_refs)


def _write_chunks(gn, width, widths, outs):
    lo = 0
    for w, oref in zip(widths, outs):
        hi = min(lo + w, width)
        blk = gn[:, lo:hi]
        if hi - lo < w:
            blk = jnp.concatenate(
                [blk, jnp.zeros((BN, w - (hi - lo)), jnp.float32)], axis=1)
        oref[...] = blk
        lo += w


def _dense_body(pieces, width_in, width_out, last, *refs):
    nr = sum(len(p) for p in pieces)
    nci = len(pieces)
    rrefs = refs[:nr]
    gs = refs[nr:nr + nci]
    dinv_ref, w_ref, b_ref = refs[nr + nci:nr + nci + 3]
    outs = refs[nr + nci + 3 + (1 if last else 0):]
    cols = []
    for p in pieces:
        acc = rrefs[p[0]][...]
        for q in p[1:]:
            acc = acc + rrefs[q][...]
        cols.append(acc)
    r = jnp.concatenate(cols, axis=1)[:, :width_in]
    g = jnp.concatenate([ref[...] for ref in gs], axis=1)[:, :width_in]
    dinv = dinv_ref[...]
    ax = dinv * (r + g)
    h = jnp.maximum(jnp.dot(ax, w_ref[...],
                            preferred_element_type=jnp.float32) + b_ref[...], 0.0)
    if last:
        batch_ref = refs[nr + nci + 3]
        h3 = jnp.concatenate(
            [h, jnp.ones((BN, 1), jnp.float32),
             jnp.zeros((BN, 320 - width_out - 1), jnp.float32)], axis=1)
        onehot = (batch_ref[...] ==
                  lax.broadcasted_iota(jnp.int32, (1, 1024), 1)
                  ).astype(jnp.bfloat16)
        contrib = lax.dot_general(
            onehot, h3.astype(jnp.bfloat16), (((0,), (0,)), ((), ())),
            preferred_element_type=jnp.float32)

        @pl.when(pl.program_id(0) == 0)
        def _():
            outs[0][...] = jnp.zeros((1024, 320), jnp.float32)

        outs[0][...] += contrib
    else:
        _write_chunks(dinv * h, width_out, _chunk_plan(width_out), outs)


def _dense_layer(r_parts, g_chunks, dinv, W, b, width_in, width_out,
                 last=False, batch_pad=None):
    nci = len(g_chunks)
    assert len(r_parts) == nci
    r_args, r_specs, pieces, idx = [], [], [], 0
    for arr, halves in r_parts:
        plist = []
        for h in halves:
            r_args.append(arr)
            r_specs.append(_rows_spec(arr.shape[1], half=h))
            plist.append(idx)
            idx += 1
        pieces.append(plist)
    if last:
        out_shape = [_f32(1024, 320)]
        out_specs = [_full_spec((1024, 320))]
    else:
        plan = _chunk_plan(width_out)
        out_shape = [_f32(NP, w) for w in plan]
        out_specs = [_rows_spec(w) for w in plan]
    in_specs = (
        r_specs
        + [_rows_spec(g.shape[1]) for g in g_chunks]
        + [_rows_spec(1), _full_spec(W.shape), _full_spec((1, width_out))]
        + ([_rows_spec(1)] if last else [])
    )
    body = functools.partial(_dense_body, pieces, width_in, width_out, last)
    args = r_args + list(g_chunks) + [dinv, W, b.reshape(1, -1)]
    if last:
        args.append(batch_pad)
    outs = pl.pallas_call(
        body, grid=_GRID, in_specs=in_specs, out_specs=out_specs,
        out_shape=out_shape,
    )(*args)
    return outs


def _head_body(pool_ref, gemb_ref, wp_ref, bp_ref, wg_ref, bg_ref,
               wf1_ref, bf1_ref, wf2_ref, bf2_ref, o_ref):
    sums = pool_ref[...]
    counts = sums[:, 312:313]
    x = sums[:, :312] / jnp.maximum(counts, 1.0)
    ge = jnp.dot(gemb_ref[...], wp_ref[...],
                 preferred_element_type=jnp.float32) + bp_ref[...]
    wg = wg_ref[...]
    gate = jax.nn.sigmoid(
        jnp.dot(x, wg[:312], preferred_element_type=jnp.float32)
        + jnp.dot(ge, wg[312:], preferred_element_type=jnp.float32)
        + bg_ref[...])
    fused = gate * ge + (1.0 - gate) * x
    h = jnp.maximum(jnp.dot(fused, wf1_ref[...],
                            preferred_element_type=jnp.float32) + bf1_ref[...], 0.0)
    o_ref[...] = jnp.dot(h, wf2_ref[...],
                         preferred_element_type=jnp.float32) + bf2_ref[...]


def _spmm_all(g_chunks, src2d, dst2d, zrows, zrows16):
    nc = len(g_chunks)
    parts = []
    k = 0
    while k + 2 <= nc:
        (r,) = _spmm2(g_chunks[k], g_chunks[k + 1], src2d, dst2d, zrows)
        parts.append((r, [0]))
        parts.append((r, [1]))
        k += 2
    if k < nc:
        if g_chunks[k].shape[1] == 16:
            (r,) = _spmm1_16(g_chunks[k], src2d, dst2d, zrows16)
        else:
            (r,) = _spmm1(g_chunks[k], src2d, dst2d, zrows)
        parts.append((r, [0, 1]))
    return parts


def kernel(mol_x, mol_edge_index, mol_batch, global_emb, W1, b1, W2, b2,
           W3, b3, Wp, bp, Wg, bg, Wf1, bf1, Wf2, bf2):
    src = mol_edge_index[0].astype(jnp.int32)
    dst = mol_edge_index[1].astype(jnp.int32)
    batch = mol_batch.astype(jnp.int32)

    pad_e = EP - E
    src2d = jnp.concatenate([src, jnp.full((pad_e,), N, jnp.int32)]).reshape(EROWS, 64)
    dst2d = jnp.concatenate([dst, jnp.full((pad_e,), N, jnp.int32)]).reshape(EROWS, 64)
    batch_pad = jnp.concatenate(
        [batch, jnp.full((NP - N,), 1024, jnp.int32)]).reshape(NP, 1)
    x_pad = jnp.concatenate([mol_x, jnp.zeros((NP - N, 78), jnp.float32)])
    ones64 = jnp.ones((64, DW), jnp.float32)
    zcol = jnp.zeros((NP, DW), jnp.float32)
    zrows = jnp.zeros((NP, CW), jnp.float32)
    zrows16 = jnp.zeros((NP, 16), jnp.float32)

    degp = _deg_kernel(dst2d, ones64, zcol)
    prep = pl.pallas_call(
        _prep_body, grid=_GRID,
        in_specs=[_rows_spec(78), _rows_spec(DW, half=0), _rows_spec(DW, half=1)],
        out_specs=[_rows_spec(1)] + [_rows_spec(w) for w in _chunk_plan(78)],
        out_shape=[_f32(NP, 1)] + [_f32(NP, w) for w in _chunk_plan(78)],
    )(x_pad, degp, degp)
    dinv, g1 = prep[0], prep[1:]

    r1 = _spmm_all(g1, src2d, dst2d, zrows, zrows16)
    g2 = _dense_layer(r1, g1, dinv, W1, b1, 78, 78)

    r2 = _spmm_all(g2, src2d, dst2d, zrows, zrows16)
    g3 = _dense_layer(r2, g2, dinv, W2, b2, 78, 156)

    r3 = _spmm_all(g3, src2d, dst2d, zrows, zrows16)
    (pool,) = _dense_layer(r3, g3, dinv, W3, b3, 156, 312, last=True,
                           batch_pad=batch_pad)

    out = pl.pallas_call(
        _head_body,
        in_specs=[_full_spec((1024, 320)), _full_spec((1024, 128)),
                  _full_spec((128, 312)), _full_spec((1, 312)),
                  _full_spec((624, 1)), _full_spec((1, 1)),
                  _full_spec((312, 1024)), _full_spec((1, 1024)),
                  _full_spec((1024, 128)), _full_spec((1, 128))],
        out_specs=_full_spec((1024, 128)),
        out_shape=_f32(1024, 128),
    )(pool, global_emb, Wp, bp.reshape(1, -1), Wg, bg.reshape(1, -1),
      Wf1, bf1.reshape(1, -1), Wf2, bf2.reshape(1, -1))
    return out
